# baseline (speedup 1.0000x reference)
import functools

import jax
import jax.numpy as jnp
from jax import lax
from jax.experimental import pallas as pl
from jax.experimental.pallas import tpu as pltpu
from jax.experimental.pallas import tpu_sc as plsc

VOCAB = 1000000
EMB = 64
B = 4096
L = 50

_NC = 2
_NS = 16
_NW = _NC * _NS

_TOTAL = B * L
_PER_W = _TOTAL // _NW
_CHUNK = 128
_NCHUNKS = _PER_W // _CHUNK


def _gather_kernel(table_hbm, idx_hbm, out_hbm, idx_v, rows_v, sem):
    wid = lax.axis_index("s") * _NC + lax.axis_index("c")
    base = wid * _PER_W
    pltpu.sync_copy(idx_hbm.at[pl.ds(base, _PER_W)], idx_v)

    def step(j, carry):
        off = j * _CHUNK
        pltpu.async_copy(
            table_hbm.at[idx_v.at[pl.ds(off, _CHUNK)]], rows_v, sem
        ).wait()
        pltpu.sync_copy(rows_v, out_hbm.at[pl.ds(base + off, _CHUNK)])
        return carry

    lax.fori_loop(0, _NCHUNKS, step, 0)


@jax.jit
def kernel(x, table):
    idx = x.reshape(_TOTAL).astype(jnp.int32)
    mesh = plsc.VectorSubcoreMesh(core_axis_name="c", subcore_axis_name="s")
    out = pl.kernel(
        _gather_kernel,
        out_type=jax.ShapeDtypeStruct((_TOTAL, EMB), jnp.float32),
        mesh=mesh,
        scratch_types=[
            pltpu.VMEM((_PER_W,), jnp.int32),
            pltpu.VMEM((_CHUNK, EMB), jnp.float32),
            pltpu.SemaphoreType.DMA,
        ],
        compiler_params=pltpu.CompilerParams(use_tc_tiling_on_sc=False),
    )(table, idx)
    return out.reshape(B, L, EMB)


# speedup vs baseline: 1.0498x; 1.0498x over previous
import functools

import jax
import jax.numpy as jnp
from jax import lax
from jax.experimental import pallas as pl
from jax.experimental.pallas import tpu as pltpu
from jax.experimental.pallas import tpu_sc as plsc

VOCAB = 1000000
EMB = 64
B = 4096
L = 50

_NC = 2
_NS = 16
_NW = _NC * _NS

_TOTAL = B * L
_PER_W = _TOTAL // _NW
_CHUNK = 128
_NCHUNKS = _PER_W // _CHUNK


_NBUF = 8
_LAG = 2


def _gather_kernel(table_hbm, idx_hbm, out_hbm, idx_v, rows_v, sem_g, sem_s):
    wid = lax.axis_index("s") * _NC + lax.axis_index("c")
    base = wid * _PER_W
    pltpu.sync_copy(idx_hbm.at[pl.ds(base, _PER_W)], idx_v)

    def fire_gather(c):
        pltpu.async_copy(
            table_hbm.at[idx_v.at[pl.ds(c * _CHUNK, _CHUNK)]],
            rows_v.at[c % _NBUF],
            sem_g,
        )

    def fire_store(c):
        pltpu.async_copy(
            rows_v.at[c % _NBUF], out_hbm.at[pl.ds(base + c * _CHUNK, _CHUNK)],
            sem_s,
        )

    def wait_gather():
        pltpu.make_async_copy(
            table_hbm.at[pl.ds(0, _CHUNK)], rows_v.at[0], sem_g
        ).wait()

    def wait_store():
        pltpu.make_async_copy(
            rows_v.at[0], out_hbm.at[pl.ds(base, _CHUNK)], sem_s
        ).wait()

    for c in range(_NBUF):
        fire_gather(c)

    def step(j, carry):
        wait_gather()
        fire_store(j)
        @pl.when(j >= _LAG)
        def _():
            wait_store()

            @pl.when(j + _NBUF - _LAG < _NCHUNKS)
            def _():
                fire_gather(j + _NBUF - _LAG)

        return carry

    lax.fori_loop(0, _NCHUNKS, step, 0)
    for _ in range(_LAG):
        wait_store()


@jax.jit
def kernel(x, table):
    idx = x.reshape(_TOTAL).astype(jnp.int32)
    mesh = plsc.VectorSubcoreMesh(core_axis_name="c", subcore_axis_name="s")
    out = pl.kernel(
        _gather_kernel,
        out_type=jax.ShapeDtypeStruct((_TOTAL, EMB), jnp.float32),
        mesh=mesh,
        scratch_types=[
            pltpu.VMEM((_PER_W,), jnp.int32),
            pltpu.VMEM((_NBUF, _CHUNK, EMB), jnp.float32),
            pltpu.SemaphoreType.DMA,
            pltpu.SemaphoreType.DMA,
        ],
        compiler_params=pltpu.CompilerParams(use_tc_tiling_on_sc=False),
    )(table, idx)
    return out.reshape(B, L, EMB)


# speedup vs baseline: 1.2184x; 1.1606x over previous
import jax
import jax.numpy as jnp
from jax import lax
from jax.experimental import pallas as pl
from jax.experimental.pallas import tpu as pltpu
from jax.experimental.pallas import tpu_sc as plsc

VOCAB = 1000000
EMB = 64
B = 4096
L = 50

_NC = 2
_NS = 16
_NW = _NC * _NS

_BPW = B // _NW
_LP = 56

_NBUF = 8
_LAG = 2


def _gather_kernel(table_hbm, x_hbm, out_hbm, idx_v, rows_v, sem_g, sem_s):
    wid = lax.axis_index("s") * _NC + lax.axis_index("c")
    b0 = wid * _BPW
    pltpu.sync_copy(x_hbm.at[pl.ds(b0, _BPW)], idx_v)

    def fire_gather(j):
        pltpu.async_copy(
            table_hbm.at[idx_v.at[j]], rows_v.at[j % _NBUF], sem_g
        )

    def fire_store(j):
        pltpu.async_copy(
            rows_v.at[j % _NBUF], out_hbm.at[b0 + j, pl.ds(0, L)], sem_s
        )

    def wait_gather():
        pltpu.make_async_copy(
            table_hbm.at[pl.ds(0, L)], rows_v.at[0], sem_g
        ).wait()

    def wait_store():
        pltpu.make_async_copy(
            rows_v.at[0], out_hbm.at[0, pl.ds(0, L)], sem_s
        ).wait()

    for j in range(_NBUF):
        fire_gather(j)

    def step(j, carry):
        wait_gather()
        fire_store(j)
        @pl.when(j >= _LAG)
        def _():
            wait_store()

            @pl.when(j + _NBUF - _LAG < _BPW)
            def _():
                fire_gather(j + _NBUF - _LAG)

        return carry

    lax.fori_loop(0, _BPW, step, 0)
    for _ in range(_LAG):
        wait_store()


@jax.jit
def kernel(x, table):
    tpad = jnp.pad(table, ((0, 0), (0, 128 - EMB)))
    mesh = plsc.VectorSubcoreMesh(core_axis_name="c", subcore_axis_name="s")
    out = pl.kernel(
        _gather_kernel,
        out_type=jax.ShapeDtypeStruct((B, _LP, 128), jnp.float32),
        mesh=mesh,
        scratch_types=[
            pltpu.VMEM((_BPW, L), jnp.int32),
            pltpu.VMEM((_NBUF, L, 128), jnp.float32),
            pltpu.SemaphoreType.DMA,
            pltpu.SemaphoreType.DMA,
        ],
        compiler_params=pltpu.CompilerParams(use_tc_tiling_on_sc=False),
    )(tpad, x.astype(jnp.int32))
    return out[:, :L, :EMB]


# speedup vs baseline: 1.2895x; 1.0584x over previous
import jax
import jax.numpy as jnp
from jax import lax
from jax.experimental import pallas as pl
from jax.experimental.pallas import tpu as pltpu
from jax.experimental.pallas import tpu_sc as plsc

VOCAB = 1000000
EMB = 64
B = 4096
L = 50

_NC = 2
_NS = 16
_NW = _NC * _NS

_BPW = B // _NW
_LP = 56

_NBUF = 8
_LAG = 2


def _gather_kernel(table_hbm, x_hbm, out_hbm, idx_v, rows_v, sem_g, sem_s):
    wid = lax.axis_index("s") * _NC + lax.axis_index("c")
    b0 = wid * _BPW
    pltpu.sync_copy(x_hbm.at[pl.ds(b0, _BPW)], idx_v)

    def fire_gather(j):
        pltpu.async_copy(
            table_hbm.at[idx_v.at[j]], rows_v.at[j % _NBUF], sem_g
        )

    def fire_store(j):
        pltpu.async_copy(
            rows_v.at[j % _NBUF],
            out_hbm.at[b0 + j, pl.ds(0, L), pl.ds(0, EMB)],
            sem_s,
        )

    def wait_gather():
        pltpu.make_async_copy(
            table_hbm.at[pl.ds(0, L)], rows_v.at[0], sem_g
        ).wait()

    def wait_store():
        pltpu.make_async_copy(
            rows_v.at[0], out_hbm.at[0, pl.ds(0, L), pl.ds(0, EMB)], sem_s
        ).wait()

    for j in range(_NBUF):
        fire_gather(j)

    def step(j, carry):
        wait_gather()
        fire_store(j)
        @pl.when(j >= _LAG)
        def _():
            wait_store()

            @pl.when(j + _NBUF - _LAG < _BPW)
            def _():
                fire_gather(j + _NBUF - _LAG)

        return carry

    lax.fori_loop(0, _BPW, step, 0)
    for _ in range(_LAG):
        wait_store()


@jax.jit
def kernel(x, table):
    tpad = jnp.pad(table, ((0, 0), (0, 128 - EMB))).reshape(2 * VOCAB, EMB)
    idx2 = x.astype(jnp.int32) * 2
    mesh = plsc.VectorSubcoreMesh(core_axis_name="c", subcore_axis_name="s")
    out = pl.kernel(
        _gather_kernel,
        out_type=jax.ShapeDtypeStruct((B, _LP, 128), jnp.float32),
        mesh=mesh,
        scratch_types=[
            pltpu.VMEM((_BPW, L), jnp.int32),
            pltpu.VMEM((_NBUF, L, EMB), jnp.float32),
            pltpu.SemaphoreType.DMA,
            pltpu.SemaphoreType.DMA,
        ],
        compiler_params=pltpu.CompilerParams(use_tc_tiling_on_sc=False),
    )(tpad, idx2)
    return out[:, :L, :EMB]
